# baseline (speedup 1.0000x reference)
import functools

import jax
import jax.numpy as jnp
from jax import lax
from jax.experimental import pallas as pl
from jax.experimental.pallas import tpu as pltpu
from jax.experimental.pallas import tpu_sc as plsc

RES = 32
GRID_C = 256
POINT_C = 128
OUT_C = 512

N_PAD = 100352
PER_WORKER = N_PAD // 32
CHUNK = 16
N_CHUNKS = PER_WORKER // CHUNK
WORDS = GRID_C // 2

_CORNER_OFF = (0, 1, 32, 33, 1024, 1025, 1056, 1057)


def _floor_clamp(t):
    i = t.astype(jnp.int32)
    f = i.astype(jnp.float32)
    i = jnp.where(f > t, i - 1, i)
    return jnp.minimum(jnp.maximum(i, 0), RES - 2)


@functools.cache
def _make_sc_gather(npts=N_PAD):
    mesh = plsc.VectorSubcoreMesh(core_axis_name="c", subcore_axis_name="s")
    PER_WORKER = npts // 32
    N_CHUNKS = PER_WORKER // CHUNK
    assert N_CHUNKS % 2 == 0 and (PER_WORKER * 3) % 8 == 0

    @functools.partial(
        pl.kernel,
        mesh=mesh,
        out_type=jax.ShapeDtypeStruct((npts, GRID_C), jnp.float32),
        scratch_types=[
            pltpu.VMEM((PER_WORKER,), jnp.float32),
            pltpu.VMEM((PER_WORKER,), jnp.float32),
            pltpu.VMEM((PER_WORKER,), jnp.float32),
            pltpu.VMEM((8 * CHUNK,), jnp.int32),
            pltpu.VMEM((8 * CHUNK,), jnp.int32),
            pltpu.VMEM((8 * CHUNK,), jnp.float32),
            pltpu.VMEM((8 * CHUNK,), jnp.float32),
            pltpu.VMEM((8 * CHUNK, WORDS), jnp.float32),
            pltpu.VMEM((8 * CHUNK, WORDS), jnp.float32),
            pltpu.VMEM((CHUNK, GRID_C), jnp.float32),
            pltpu.VMEM((CHUNK, GRID_C), jnp.float32),
            pltpu.SemaphoreType.DMA,
            pltpu.SemaphoreType.DMA,
            pltpu.SemaphoreType.DMA,
            pltpu.SemaphoreType.DMA,
        ],
    )
    def sc_gather(table_hbm, xs_hbm, ys_hbm, zs_hbm, out_hbm, xbuf, ybuf,
                  zbuf, idx_a, idx_b, wts_a, wts_b, rows_a, rows_b, out_a,
                  out_b, sem_a, sem_b, osem_a, osem_b):
        cid = lax.axis_index("c")
        sid = lax.axis_index("s")
        wid = sid * 2 + cid
        base = pl.multiple_of(wid * PER_WORKER, 8)
        pltpu.sync_copy(xs_hbm.at[pl.ds(base, PER_WORKER)], xbuf)
        pltpu.sync_copy(ys_hbm.at[pl.ds(base, PER_WORKER)], ybuf)
        pltpu.sync_copy(zs_hbm.at[pl.ds(base, PER_WORKER)], zbuf)

        zeros16 = jnp.zeros((16,), jnp.int32)
        gdn = lax.GatherDimensionNumbers(
            offset_dims=(), collapsed_slice_dims=(0,), start_index_map=(0,))

        def build(g, idx_v, wts_v):
            off = pl.multiple_of(g * CHUNK, 8)
            vx = xbuf[pl.ds(off, CHUNK)]
            vy = ybuf[pl.ds(off, CHUNK)]
            vz = zbuf[pl.ds(off, CHUNK)]

            x31 = vx * jnp.float32(RES - 1)
            y31 = vy * jnp.float32(RES - 1)
            z31 = vz * jnp.float32(RES - 1)
            x0 = _floor_clamp(x31)
            y0 = _floor_clamp(y31)
            z0 = _floor_clamp(z31)
            wx1 = x31 - x0.astype(jnp.float32)
            wy1 = y31 - y0.astype(jnp.float32)
            wz1 = z31 - z0.astype(jnp.float32)
            wx0 = 1.0 - wx1
            wy0 = 1.0 - wy1
            wz0 = 1.0 - wz1

            w00 = wz0 * wy0
            w01 = wz0 * wy1
            w10 = wz1 * wy0
            w11 = wz1 * wy1
            for ci, w in enumerate((w00 * wx0, w00 * wx1, w01 * wx0,
                                    w01 * wx1, w10 * wx0, w10 * wx1,
                                    w11 * wx0, w11 * wx1)):
                wts_v[pl.ds(ci * CHUNK, CHUNK)] = w

            r = z0 * (RES * RES) + y0 * RES + x0
            for ci in range(8):
                idx_v[pl.ds(ci * CHUNK, CHUNK)] = r + _CORNER_OFF[ci]

        def fire(idx_v, rows_v, sem):
            pltpu.async_copy(table_hbm.at[idx_v], rows_v, sem)

        def wait(idx_v, rows_v, sem):
            pltpu.make_async_copy(table_hbm.at[idx_v], rows_v, sem).wait()

        def _oslice(g):
            return out_hbm.at[pl.ds(pl.multiple_of(base + g * CHUNK, 8),
                                    CHUNK)]

        def compute_store(g, rows_v, wts_v, out_v, osem, drain_pred):
            wrows = [wts_v[pl.ds(ci * CHUNK, CHUNK)] for ci in range(8)]
            i32 = jnp.int32
            f32 = jnp.float32

            @pl.when(drain_pred)
            def _():
                pltpu.make_async_copy(out_v, _oslice(g), osem).wait()

            @plsc.parallel_loop(0, CHUNK, unroll=2)
            def pt_body(l):
                lsplat = (zeros16 + l)[:, None]
                wvec = [lax.gather(
                            wrows[ci], lsplat, gdn, (1,),
                            mode=lax.GatherScatterMode.PROMISE_IN_BOUNDS)
                        for ci in range(8)]
                for sgi in range(WORDS // 16):
                    acc_a = None
                    acc_b = None
                    for ci in range(8):
                        seg = rows_v[ci * CHUNK + l, pl.ds(sgi * 16, 16)]
                        w_i = lax.bitcast_convert_type(seg, i32)
                        a = lax.bitcast_convert_type(
                            lax.shift_left(w_i, 16), f32)
                        bb = seg
                        ta = a * wvec[ci]
                        tb = bb * wvec[ci]
                        acc_a = ta if acc_a is None else acc_a + ta
                        acc_b = tb if acc_b is None else acc_b + tb
                    out_v[l, pl.ds(sgi * 16, 16)] = acc_a
                    out_v[l, pl.ds(WORDS + sgi * 16, 16)] = acc_b

            pltpu.async_copy(out_v, _oslice(g), osem)

        build(0, idx_a, wts_a)
        fire(idx_a, rows_a, sem_a)

        def pair_body(i, carry):
            g0 = i * 2
            build(g0 + 1, idx_b, wts_b)
            fire(idx_b, rows_b, sem_b)
            wait(idx_a, rows_a, sem_a)
            compute_store(g0, rows_a, wts_a, out_a, osem_a, i > 0)

            @pl.when(i < N_CHUNKS // 2 - 1)
            def _():
                build(g0 + 2, idx_a, wts_a)
                fire(idx_a, rows_a, sem_a)

            wait(idx_b, rows_b, sem_b)
            compute_store(g0 + 1, rows_b, wts_b, out_b, osem_b, i > 0)
            return carry

        lax.fori_loop(0, N_CHUNKS // 2, pair_body, 0)
        pltpu.make_async_copy(out_a, _oslice(N_CHUNKS - 2), osem_a).wait()
        pltpu.make_async_copy(out_b, _oslice(N_CHUNKS - 1), osem_b).wait()

    return sc_gather


_ROWS_BLK = 512


def _tc_body(pf_ref, samp_ref, wp_ref, wg_ref, b_ref, gamma_ref, beta_ref,
             out_ref):
    y = jnp.dot(pf_ref[...].astype(jnp.bfloat16), wp_ref[...],
                preferred_element_type=jnp.float32)
    y = y + jnp.dot(samp_ref[...].astype(jnp.bfloat16), wg_ref[...],
                    preferred_element_type=jnp.float32)
    y = y + b_ref[...]
    mu = jnp.mean(y, axis=1, keepdims=True)
    yc = y - mu
    var = jnp.mean(yc * yc, axis=1, keepdims=True)
    out_ref[...] = yc * lax.rsqrt(var + 1e-5) * gamma_ref[...] + beta_ref[...]


def _tc_fused(pf, samp, wp, wg, b, gamma, beta):
    n = pf.shape[0]
    n_blocks = pl.cdiv(n, _ROWS_BLK)
    return pl.pallas_call(
        _tc_body,
        grid=(n_blocks,),
        in_specs=[
            pl.BlockSpec((_ROWS_BLK, POINT_C), lambda i: (i, 0)),
            pl.BlockSpec((_ROWS_BLK, GRID_C), lambda i: (i, 0)),
            pl.BlockSpec((POINT_C, OUT_C), lambda i: (0, 0)),
            pl.BlockSpec((GRID_C, OUT_C), lambda i: (0, 0)),
            pl.BlockSpec((1, OUT_C), lambda i: (0, 0)),
            pl.BlockSpec((1, OUT_C), lambda i: (0, 0)),
            pl.BlockSpec((1, OUT_C), lambda i: (0, 0)),
        ],
        out_specs=pl.BlockSpec((_ROWS_BLK, OUT_C), lambda i: (i, 0)),
        out_shape=jax.ShapeDtypeStruct((n, OUT_C), jnp.float32),
    )(pf, samp, wp, wg, b, gamma, beta)


def kernel(grid_features, vertices, point_feats, W, b, gamma, beta):
    n = vertices.shape[0]
    table = grid_features[0].reshape(GRID_C, RES * RES * RES).T
    table_w = lax.bitcast_convert_type(
        table.astype(jnp.bfloat16).reshape(-1, WORDS, 2), jnp.float32)
    vpad = jnp.pad(vertices, ((0, N_PAD - n), (0, 0)))
    xs, ys, zs = vpad[:, 0], vpad[:, 1], vpad[:, 2]
    wb = W.astype(jnp.bfloat16)
    perm = jnp.asarray(list(range(0, GRID_C, 2)) + list(range(1, GRID_C, 2)))
    wg = wb[POINT_C:][perm]
    wp = wb[:POINT_C]
    args = (b[None, :], gamma[None, :], beta[None, :])

    half = N_PAD // 2
    sc = _make_sc_gather(half)
    samp0 = sc(table_w, xs[:half], ys[:half], zs[:half])
    samp1 = sc(table_w, xs[half:], ys[half:], zs[half:])
    out0 = _tc_fused(point_feats[:half], samp0, wp, wg, *args)
    out1 = _tc_fused(point_feats[half:n], samp1, wp, wg, *args)
    return jnp.concatenate([out0, out1], axis=0)


# speedup vs baseline: 1.1079x; 1.1079x over previous
import functools

import jax
import jax.numpy as jnp
from jax import lax
from jax.experimental import pallas as pl
from jax.experimental.pallas import tpu as pltpu
from jax.experimental.pallas import tpu_sc as plsc

RES = 32
GRID_C = 256
POINT_C = 128
OUT_C = 512

N_PAD = 100352
PER_WORKER = N_PAD // 32
CHUNK = 16
N_CHUNKS = PER_WORKER // CHUNK
WORDS = GRID_C // 2

_CORNER_OFF = (0, 1, 32, 33, 1024, 1025, 1056, 1057)


def _floor_clamp(t):
    i = t.astype(jnp.int32)
    f = i.astype(jnp.float32)
    i = jnp.where(f > t, i - 1, i)
    return jnp.minimum(jnp.maximum(i, 0), RES - 2)


@functools.cache
def _make_sc_gather(npts=N_PAD):
    mesh = plsc.VectorSubcoreMesh(core_axis_name="c", subcore_axis_name="s")
    PER_WORKER = npts // 32
    N_CHUNKS = PER_WORKER // CHUNK
    assert N_CHUNKS % 2 == 0 and (PER_WORKER * 3) % 8 == 0

    @functools.partial(
        pl.kernel,
        mesh=mesh,
        out_type=jax.ShapeDtypeStruct((npts, GRID_C), jnp.float32),
        scratch_types=[
            pltpu.VMEM((PER_WORKER,), jnp.float32),
            pltpu.VMEM((PER_WORKER,), jnp.float32),
            pltpu.VMEM((PER_WORKER,), jnp.float32),
            pltpu.VMEM((8 * CHUNK,), jnp.int32),
            pltpu.VMEM((8 * CHUNK,), jnp.int32),
            pltpu.VMEM((8 * CHUNK,), jnp.float32),
            pltpu.VMEM((8 * CHUNK,), jnp.float32),
            pltpu.VMEM((8 * CHUNK, WORDS), jnp.float32),
            pltpu.VMEM((8 * CHUNK, WORDS), jnp.float32),
            pltpu.VMEM((CHUNK, GRID_C), jnp.float32),
            pltpu.VMEM((CHUNK, GRID_C), jnp.float32),
            pltpu.SemaphoreType.DMA,
            pltpu.SemaphoreType.DMA,
            pltpu.SemaphoreType.DMA,
            pltpu.SemaphoreType.DMA,
        ],
    )
    def sc_gather(table_hbm, vflat_hbm, out_hbm, xbuf, ybuf,
                  zbuf, idx_a, idx_b, wts_a, wts_b, rows_a, rows_b, out_a,
                  out_b, sem_a, sem_b, osem_a, osem_b):
        cid = lax.axis_index("c")
        sid = lax.axis_index("s")
        wid = sid * 2 + cid
        base = pl.multiple_of(wid * PER_WORKER, 8)
        pltpu.sync_copy(vflat_hbm.at[pl.ds(base, PER_WORKER)], xbuf)
        pltpu.sync_copy(
            vflat_hbm.at[pl.ds(pl.multiple_of(npts + base, 8), PER_WORKER)],
            ybuf)
        pltpu.sync_copy(
            vflat_hbm.at[pl.ds(pl.multiple_of(2 * npts + base, 8),
                               PER_WORKER)],
            zbuf)

        zeros16 = jnp.zeros((16,), jnp.int32)
        gdn = lax.GatherDimensionNumbers(
            offset_dims=(), collapsed_slice_dims=(0,), start_index_map=(0,))

        def build(g, idx_v, wts_v):
            off = pl.multiple_of(g * CHUNK, 8)
            vx = xbuf[pl.ds(off, CHUNK)]
            vy = ybuf[pl.ds(off, CHUNK)]
            vz = zbuf[pl.ds(off, CHUNK)]

            x31 = vx * jnp.float32(RES - 1)
            y31 = vy * jnp.float32(RES - 1)
            z31 = vz * jnp.float32(RES - 1)
            x0 = _floor_clamp(x31)
            y0 = _floor_clamp(y31)
            z0 = _floor_clamp(z31)
            wx1 = x31 - x0.astype(jnp.float32)
            wy1 = y31 - y0.astype(jnp.float32)
            wz1 = z31 - z0.astype(jnp.float32)
            wx0 = 1.0 - wx1
            wy0 = 1.0 - wy1
            wz0 = 1.0 - wz1

            w00 = wz0 * wy0
            w01 = wz0 * wy1
            w10 = wz1 * wy0
            w11 = wz1 * wy1
            for ci, w in enumerate((w00 * wx0, w00 * wx1, w01 * wx0,
                                    w01 * wx1, w10 * wx0, w10 * wx1,
                                    w11 * wx0, w11 * wx1)):
                wts_v[pl.ds(ci * CHUNK, CHUNK)] = w

            r = z0 * (RES * RES) + y0 * RES + x0
            for ci in range(8):
                idx_v[pl.ds(ci * CHUNK, CHUNK)] = r + _CORNER_OFF[ci]

        def fire(idx_v, rows_v, sem):
            pltpu.async_copy(table_hbm.at[idx_v], rows_v, sem)

        def wait(idx_v, rows_v, sem):
            pltpu.make_async_copy(table_hbm.at[idx_v], rows_v, sem).wait()

        def _oslice(g):
            return out_hbm.at[pl.ds(pl.multiple_of(base + g * CHUNK, 8),
                                    CHUNK)]

        def compute_store(g, rows_v, wts_v, out_v, osem, drain_pred):
            wrows = [wts_v[pl.ds(ci * CHUNK, CHUNK)] for ci in range(8)]
            i32 = jnp.int32
            f32 = jnp.float32

            @pl.when(drain_pred)
            def _():
                pltpu.make_async_copy(out_v, _oslice(g), osem).wait()

            @plsc.parallel_loop(0, CHUNK, unroll=2)
            def pt_body(l):
                lsplat = (zeros16 + l)[:, None]
                wvec = [lax.gather(
                            wrows[ci], lsplat, gdn, (1,),
                            mode=lax.GatherScatterMode.PROMISE_IN_BOUNDS)
                        for ci in range(8)]
                for sgi in range(WORDS // 16):
                    acc_a = None
                    acc_b = None
                    for ci in range(8):
                        seg = rows_v[ci * CHUNK + l, pl.ds(sgi * 16, 16)]
                        w_i = lax.bitcast_convert_type(seg, i32)
                        a = lax.bitcast_convert_type(
                            lax.shift_left(w_i, 16), f32)
                        bb = seg
                        ta = a * wvec[ci]
                        tb = bb * wvec[ci]
                        acc_a = ta if acc_a is None else acc_a + ta
                        acc_b = tb if acc_b is None else acc_b + tb
                    out_v[l, pl.ds(sgi * 16, 16)] = acc_a
                    out_v[l, pl.ds(WORDS + sgi * 16, 16)] = acc_b

            pltpu.async_copy(out_v, _oslice(g), osem)

        build(0, idx_a, wts_a)
        fire(idx_a, rows_a, sem_a)

        def pair_body(i, carry):
            g0 = i * 2
            build(g0 + 1, idx_b, wts_b)
            fire(idx_b, rows_b, sem_b)
            wait(idx_a, rows_a, sem_a)
            compute_store(g0, rows_a, wts_a, out_a, osem_a, i > 0)

            @pl.when(i < N_CHUNKS // 2 - 1)
            def _():
                build(g0 + 2, idx_a, wts_a)
                fire(idx_a, rows_a, sem_a)

            wait(idx_b, rows_b, sem_b)
            compute_store(g0 + 1, rows_b, wts_b, out_b, osem_b, i > 0)
            return carry

        lax.fori_loop(0, N_CHUNKS // 2, pair_body, 0)
        pltpu.make_async_copy(out_a, _oslice(N_CHUNKS - 2), osem_a).wait()
        pltpu.make_async_copy(out_b, _oslice(N_CHUNKS - 1), osem_b).wait()

    return sc_gather


_ROWS_BLK = 512


def _tc_body(pf_ref, samp_ref, wp_ref, wg_ref, b_ref, gamma_ref, beta_ref,
             out_ref):
    y = jnp.dot(pf_ref[...].astype(jnp.bfloat16), wp_ref[...],
                preferred_element_type=jnp.float32)
    y = y + jnp.dot(samp_ref[...].astype(jnp.bfloat16), wg_ref[...],
                    preferred_element_type=jnp.float32)
    y = y + b_ref[...]
    mu = jnp.mean(y, axis=1, keepdims=True)
    yc = y - mu
    var = jnp.mean(yc * yc, axis=1, keepdims=True)
    out_ref[...] = yc * lax.rsqrt(var + 1e-5) * gamma_ref[...] + beta_ref[...]


def _tc_fused(pf, samp, wp, wg, b, gamma, beta):
    n = pf.shape[0]
    n_blocks = pl.cdiv(n, _ROWS_BLK)
    return pl.pallas_call(
        _tc_body,
        grid=(n_blocks,),
        in_specs=[
            pl.BlockSpec((_ROWS_BLK, POINT_C), lambda i: (i, 0)),
            pl.BlockSpec((_ROWS_BLK, GRID_C), lambda i: (i, 0)),
            pl.BlockSpec((POINT_C, OUT_C), lambda i: (0, 0)),
            pl.BlockSpec((GRID_C, OUT_C), lambda i: (0, 0)),
            pl.BlockSpec((1, OUT_C), lambda i: (0, 0)),
            pl.BlockSpec((1, OUT_C), lambda i: (0, 0)),
            pl.BlockSpec((1, OUT_C), lambda i: (0, 0)),
        ],
        out_specs=pl.BlockSpec((_ROWS_BLK, OUT_C), lambda i: (i, 0)),
        out_shape=jax.ShapeDtypeStruct((n, OUT_C), jnp.float32),
    )(pf, samp, wp, wg, b, gamma, beta)


def kernel(grid_features, vertices, point_feats, W, b, gamma, beta):
    n = vertices.shape[0]
    table = grid_features[0].reshape(GRID_C, RES * RES * RES).T
    table_w = lax.bitcast_convert_type(
        table.astype(jnp.bfloat16).reshape(-1, WORDS, 2), jnp.float32)
    vflat = jnp.pad(vertices, ((0, N_PAD - n), (0, 0))).T.reshape(-1)
    wb = W.astype(jnp.bfloat16)
    perm = jnp.asarray(list(range(0, GRID_C, 2)) + list(range(1, GRID_C, 2)))
    wg = wb[POINT_C:][perm]
    samp_w = _make_sc_gather()(table_w, vflat)
    return _tc_fused(point_feats, samp_w, wb[:POINT_C], wg,
                     b[None, :], gamma[None, :], beta[None, :])


# speedup vs baseline: 1.2123x; 1.0943x over previous
import functools

import jax
import jax.numpy as jnp
from jax import lax
from jax.experimental import pallas as pl
from jax.experimental.pallas import tpu as pltpu
from jax.experimental.pallas import tpu_sc as plsc

RES = 32
GRID_C = 256
POINT_C = 128
OUT_C = 512

N_PAD = 100352
PER_WORKER = N_PAD // 32
CHUNK = 16
N_CHUNKS = PER_WORKER // CHUNK
WORDS = GRID_C // 2

_CORNER_OFF = (0, 1, 32, 33, 1024, 1025, 1056, 1057)


def _floor_clamp(t):
    i = t.astype(jnp.int32)
    f = i.astype(jnp.float32)
    i = jnp.where(f > t, i - 1, i)
    return jnp.minimum(jnp.maximum(i, 0), RES - 2)


@functools.cache
def _make_sc_gather(npts=N_PAD):
    mesh = plsc.VectorSubcoreMesh(core_axis_name="c", subcore_axis_name="s")
    PER_WORKER = npts // 32
    N_CHUNKS = PER_WORKER // CHUNK
    assert N_CHUNKS % 2 == 0 and (PER_WORKER * 3) % 8 == 0

    @functools.partial(
        pl.kernel,
        mesh=mesh,
        out_type=jax.ShapeDtypeStruct((npts, GRID_C), jnp.float32),
        scratch_types=[
            pltpu.VMEM((PER_WORKER,), jnp.float32),
            pltpu.VMEM((PER_WORKER,), jnp.float32),
            pltpu.VMEM((PER_WORKER,), jnp.float32),
            pltpu.VMEM((8 * CHUNK,), jnp.int32),
            pltpu.VMEM((8 * CHUNK,), jnp.int32),
            pltpu.VMEM((8 * CHUNK,), jnp.float32),
            pltpu.VMEM((8 * CHUNK,), jnp.float32),
            pltpu.VMEM((8 * CHUNK, WORDS), jnp.float32),
            pltpu.VMEM((8 * CHUNK, WORDS), jnp.float32),
            pltpu.VMEM((CHUNK, GRID_C), jnp.float32),
            pltpu.VMEM((CHUNK, GRID_C), jnp.float32),
            pltpu.SemaphoreType.DMA,
            pltpu.SemaphoreType.DMA,
            pltpu.SemaphoreType.DMA,
            pltpu.SemaphoreType.DMA,
        ],
    )
    def sc_gather(table_hbm, vflat_hbm, out_hbm, xbuf, ybuf,
                  zbuf, idx_a, idx_b, wts_a, wts_b, rows_a, rows_b, out_a,
                  out_b, sem_a, sem_b, osem_a, osem_b):
        cid = lax.axis_index("c")
        sid = lax.axis_index("s")
        wid = sid * 2 + cid
        base = pl.multiple_of(wid * PER_WORKER, 8)
        pltpu.sync_copy(vflat_hbm.at[pl.ds(base, PER_WORKER)], xbuf)
        pltpu.sync_copy(
            vflat_hbm.at[pl.ds(pl.multiple_of(npts + base, 8), PER_WORKER)],
            ybuf)
        pltpu.sync_copy(
            vflat_hbm.at[pl.ds(pl.multiple_of(2 * npts + base, 8),
                               PER_WORKER)],
            zbuf)

        zeros16 = jnp.zeros((16,), jnp.int32)
        gdn = lax.GatherDimensionNumbers(
            offset_dims=(), collapsed_slice_dims=(0,), start_index_map=(0,))

        def build(g, idx_v, wts_v):
            off = pl.multiple_of(g * CHUNK, 8)
            vx = xbuf[pl.ds(off, CHUNK)]
            vy = ybuf[pl.ds(off, CHUNK)]
            vz = zbuf[pl.ds(off, CHUNK)]

            x31 = vx * jnp.float32(RES - 1)
            y31 = vy * jnp.float32(RES - 1)
            z31 = vz * jnp.float32(RES - 1)
            x0 = _floor_clamp(x31)
            y0 = _floor_clamp(y31)
            z0 = _floor_clamp(z31)
            wx1 = x31 - x0.astype(jnp.float32)
            wy1 = y31 - y0.astype(jnp.float32)
            wz1 = z31 - z0.astype(jnp.float32)
            wx0 = 1.0 - wx1
            wy0 = 1.0 - wy1
            wz0 = 1.0 - wz1

            w00 = wz0 * wy0
            w01 = wz0 * wy1
            w10 = wz1 * wy0
            w11 = wz1 * wy1
            for ci, w in enumerate((w00 * wx0, w00 * wx1, w01 * wx0,
                                    w01 * wx1, w10 * wx0, w10 * wx1,
                                    w11 * wx0, w11 * wx1)):
                wts_v[pl.ds(ci * CHUNK, CHUNK)] = w

            r = z0 * (RES * RES) + y0 * RES + x0
            for ci in range(8):
                idx_v[pl.ds(ci * CHUNK, CHUNK)] = r + _CORNER_OFF[ci]

        def fire(idx_v, rows_v, sem):
            pltpu.async_copy(table_hbm.at[idx_v], rows_v, sem)

        def wait(idx_v, rows_v, sem):
            pltpu.make_async_copy(table_hbm.at[idx_v], rows_v, sem).wait()

        def _oslice(g):
            return out_hbm.at[pl.ds(pl.multiple_of(base + g * CHUNK, 8),
                                    CHUNK)]

        def compute_store(g, rows_v, wts_v, out_v, osem, drain_pred):
            wrows = [wts_v[pl.ds(ci * CHUNK, CHUNK)] for ci in range(8)]
            i32 = jnp.int32
            f32 = jnp.float32

            @pl.when(drain_pred)
            def _():
                pltpu.make_async_copy(out_v, _oslice(g), osem).wait()

            @plsc.parallel_loop(0, CHUNK, unroll=2)
            def pt_body(l):
                lsplat = (zeros16 + l)[:, None]
                wvec = [lax.gather(
                            wrows[ci], lsplat, gdn, (1,),
                            mode=lax.GatherScatterMode.PROMISE_IN_BOUNDS)
                        for ci in range(8)]
                for sgi in range(WORDS // 16):
                    acc_a = None
                    acc_b = None
                    for ci in range(8):
                        seg = rows_v[ci * CHUNK + l, pl.ds(sgi * 16, 16)]
                        w_i = lax.bitcast_convert_type(seg, i32)
                        a = lax.bitcast_convert_type(
                            lax.shift_left(w_i, 16), f32)
                        bb = seg
                        ta = a * wvec[ci]
                        tb = bb * wvec[ci]
                        acc_a = ta if acc_a is None else acc_a + ta
                        acc_b = tb if acc_b is None else acc_b + tb
                    out_v[l, pl.ds(sgi * 16, 16)] = acc_a
                    out_v[l, pl.ds(WORDS + sgi * 16, 16)] = acc_b

            pltpu.async_copy(out_v, _oslice(g), osem)

        build(0, idx_a, wts_a)
        fire(idx_a, rows_a, sem_a)

        def pair_body(i, carry):
            g0 = i * 2
            build(g0 + 1, idx_b, wts_b)
            fire(idx_b, rows_b, sem_b)
            wait(idx_a, rows_a, sem_a)
            compute_store(g0, rows_a, wts_a, out_a, osem_a, i > 0)

            @pl.when(i < N_CHUNKS // 2 - 1)
            def _():
                build(g0 + 2, idx_a, wts_a)
                fire(idx_a, rows_a, sem_a)

            wait(idx_b, rows_b, sem_b)
            compute_store(g0 + 1, rows_b, wts_b, out_b, osem_b, i > 0)
            return carry

        lax.fori_loop(0, N_CHUNKS // 2, pair_body, 0)
        pltpu.make_async_copy(out_a, _oslice(N_CHUNKS - 2), osem_a).wait()
        pltpu.make_async_copy(out_b, _oslice(N_CHUNKS - 1), osem_b).wait()

    return sc_gather


_ROWS_BLK = 1024


def _tc_body(pf_ref, samp_ref, wp_ref, wg_ref, b_ref, gamma_ref, beta_ref,
             out_ref):
    y = jnp.dot(pf_ref[...].astype(jnp.bfloat16), wp_ref[...],
                preferred_element_type=jnp.float32)
    y = y + jnp.dot(samp_ref[...].astype(jnp.bfloat16), wg_ref[...],
                    preferred_element_type=jnp.float32)
    y = y + b_ref[...]
    mu = jnp.mean(y, axis=1, keepdims=True)
    yc = y - mu
    var = jnp.mean(yc * yc, axis=1, keepdims=True)
    out_ref[...] = yc * lax.rsqrt(var + 1e-5) * gamma_ref[...] + beta_ref[...]


def _tc_fused(pf, samp, wp, wg, b, gamma, beta):
    n = pf.shape[0]
    n_blocks = pl.cdiv(n, _ROWS_BLK)
    return pl.pallas_call(
        _tc_body,
        grid=(n_blocks,),
        in_specs=[
            pl.BlockSpec((_ROWS_BLK, POINT_C), lambda i: (i, 0)),
            pl.BlockSpec((_ROWS_BLK, GRID_C), lambda i: (i, 0)),
            pl.BlockSpec((POINT_C, OUT_C), lambda i: (0, 0)),
            pl.BlockSpec((GRID_C, OUT_C), lambda i: (0, 0)),
            pl.BlockSpec((1, OUT_C), lambda i: (0, 0)),
            pl.BlockSpec((1, OUT_C), lambda i: (0, 0)),
            pl.BlockSpec((1, OUT_C), lambda i: (0, 0)),
        ],
        out_specs=pl.BlockSpec((_ROWS_BLK, OUT_C), lambda i: (i, 0)),
        out_shape=jax.ShapeDtypeStruct((n, OUT_C), jnp.float32),
    )(pf, samp, wp, wg, b, gamma, beta)


def kernel(grid_features, vertices, point_feats, W, b, gamma, beta):
    n = vertices.shape[0]
    table = grid_features[0].reshape(GRID_C, RES * RES * RES).T
    table_w = lax.bitcast_convert_type(
        table.astype(jnp.bfloat16).reshape(-1, WORDS, 2), jnp.float32)
    vflat = jnp.pad(vertices, ((0, N_PAD - n), (0, 0))).T.reshape(-1)
    wb = W.astype(jnp.bfloat16)
    perm = jnp.asarray(list(range(0, GRID_C, 2)) + list(range(1, GRID_C, 2)))
    wg = wb[POINT_C:][perm]
    samp_w = _make_sc_gather()(table_w, vflat)
    return _tc_fused(point_feats, samp_w, wb[:POINT_C], wg,
                     b[None, :], gamma[None, :], beta[None, :])


# speedup vs baseline: 1.2762x; 1.0527x over previous
import functools

import jax
import jax.numpy as jnp
from jax import lax
from jax.experimental import pallas as pl
from jax.experimental.pallas import tpu as pltpu
from jax.experimental.pallas import tpu_sc as plsc

RES = 32
GRID_C = 256
POINT_C = 128
OUT_C = 512

N_PAD = 100352
PER_WORKER = N_PAD // 32
CHUNK = 16
N_CHUNKS = PER_WORKER // CHUNK
WORDS = GRID_C // 2

_CORNER_OFF = (0, 1, 32, 33, 1024, 1025, 1056, 1057)


def _floor_clamp(t):
    i = t.astype(jnp.int32)
    f = i.astype(jnp.float32)
    i = jnp.where(f > t, i - 1, i)
    return jnp.minimum(jnp.maximum(i, 0), RES - 2)


@functools.cache
def _make_sc_gather(npts=N_PAD):
    mesh = plsc.VectorSubcoreMesh(core_axis_name="c", subcore_axis_name="s")
    PER_WORKER = npts // 32
    N_CHUNKS = PER_WORKER // CHUNK
    assert N_CHUNKS % 2 == 0 and (PER_WORKER * 3) % 8 == 0

    @functools.partial(
        pl.kernel,
        mesh=mesh,
        out_type=jax.ShapeDtypeStruct((npts, GRID_C), jnp.float32),
        scratch_types=[
            pltpu.VMEM((PER_WORKER,), jnp.float32),
            pltpu.VMEM((PER_WORKER,), jnp.float32),
            pltpu.VMEM((PER_WORKER,), jnp.float32),
            pltpu.VMEM((8 * CHUNK,), jnp.int32),
            pltpu.VMEM((8 * CHUNK,), jnp.int32),
            pltpu.VMEM((8 * CHUNK,), jnp.float32),
            pltpu.VMEM((8 * CHUNK,), jnp.float32),
            pltpu.VMEM((8 * CHUNK, WORDS), jnp.float32),
            pltpu.VMEM((8 * CHUNK, WORDS), jnp.float32),
            pltpu.VMEM((CHUNK, GRID_C), jnp.float32),
            pltpu.VMEM((CHUNK, GRID_C), jnp.float32),
            pltpu.SemaphoreType.DMA,
            pltpu.SemaphoreType.DMA,
            pltpu.SemaphoreType.DMA,
            pltpu.SemaphoreType.DMA,
        ],
    )
    def sc_gather(table_hbm, vflat_hbm, out_hbm, xbuf, ybuf,
                  zbuf, idx_a, idx_b, wts_a, wts_b, rows_a, rows_b, out_a,
                  out_b, sem_a, sem_b, osem_a, osem_b):
        cid = lax.axis_index("c")
        sid = lax.axis_index("s")
        wid = sid * 2 + cid
        base = pl.multiple_of(wid * PER_WORKER, 8)
        pltpu.sync_copy(vflat_hbm.at[pl.ds(base, PER_WORKER)], xbuf)
        pltpu.sync_copy(
            vflat_hbm.at[pl.ds(pl.multiple_of(npts + base, 8), PER_WORKER)],
            ybuf)
        pltpu.sync_copy(
            vflat_hbm.at[pl.ds(pl.multiple_of(2 * npts + base, 8),
                               PER_WORKER)],
            zbuf)

        zeros16 = jnp.zeros((16,), jnp.int32)
        gdn = lax.GatherDimensionNumbers(
            offset_dims=(), collapsed_slice_dims=(0,), start_index_map=(0,))

        def build(g, idx_v, wts_v):
            off = pl.multiple_of(g * CHUNK, 8)
            vx = xbuf[pl.ds(off, CHUNK)]
            vy = ybuf[pl.ds(off, CHUNK)]
            vz = zbuf[pl.ds(off, CHUNK)]

            x31 = vx * jnp.float32(RES - 1)
            y31 = vy * jnp.float32(RES - 1)
            z31 = vz * jnp.float32(RES - 1)
            x0 = _floor_clamp(x31)
            y0 = _floor_clamp(y31)
            z0 = _floor_clamp(z31)
            wx1 = x31 - x0.astype(jnp.float32)
            wy1 = y31 - y0.astype(jnp.float32)
            wz1 = z31 - z0.astype(jnp.float32)
            wx0 = 1.0 - wx1
            wy0 = 1.0 - wy1
            wz0 = 1.0 - wz1

            w00 = wz0 * wy0
            w01 = wz0 * wy1
            w10 = wz1 * wy0
            w11 = wz1 * wy1
            for ci, w in enumerate((w00 * wx0, w00 * wx1, w01 * wx0,
                                    w01 * wx1, w10 * wx0, w10 * wx1,
                                    w11 * wx0, w11 * wx1)):
                wts_v[pl.ds(ci * CHUNK, CHUNK)] = w

            r = z0 * (RES * RES) + y0 * RES + x0
            for ci in range(8):
                idx_v[pl.ds(ci * CHUNK, CHUNK)] = r + _CORNER_OFF[ci]

        def fire(idx_v, rows_v, sem):
            pltpu.async_copy(table_hbm.at[idx_v], rows_v, sem)

        def wait(idx_v, rows_v, sem):
            pltpu.make_async_copy(table_hbm.at[idx_v], rows_v, sem).wait()

        def _oslice(g):
            return out_hbm.at[pl.ds(pl.multiple_of(base + g * CHUNK, 8),
                                    CHUNK)]

        def compute_store(g, rows_v, wts_v, out_v, osem, drain_pred):
            wrows = [wts_v[pl.ds(ci * CHUNK, CHUNK)] for ci in range(8)]
            i32 = jnp.int32
            f32 = jnp.float32

            @pl.when(drain_pred)
            def _():
                pltpu.make_async_copy(out_v, _oslice(g), osem).wait()

            @plsc.parallel_loop(0, CHUNK, unroll=2)
            def pt_body(l):
                lsplat = (zeros16 + l)[:, None]
                wvec = [lax.gather(
                            wrows[ci], lsplat, gdn, (1,),
                            mode=lax.GatherScatterMode.PROMISE_IN_BOUNDS)
                        for ci in range(8)]
                for sgi in range(WORDS // 16):
                    acc_a = None
                    acc_b = None
                    for ci in range(8):
                        seg = rows_v[ci * CHUNK + l, pl.ds(sgi * 16, 16)]
                        w_i = lax.bitcast_convert_type(seg, i32)
                        a = lax.bitcast_convert_type(
                            lax.shift_left(w_i, 16), f32)
                        bb = seg
                        ta = a * wvec[ci]
                        tb = bb * wvec[ci]
                        acc_a = ta if acc_a is None else acc_a + ta
                        acc_b = tb if acc_b is None else acc_b + tb
                    out_v[l, pl.ds(sgi * 16, 16)] = acc_a
                    out_v[l, pl.ds(WORDS + sgi * 16, 16)] = acc_b

            pltpu.async_copy(out_v, _oslice(g), osem)

        build(0, idx_a, wts_a)
        fire(idx_a, rows_a, sem_a)

        def pair_body(i, carry):
            g0 = i * 2
            build(g0 + 1, idx_b, wts_b)
            fire(idx_b, rows_b, sem_b)
            wait(idx_a, rows_a, sem_a)
            compute_store(g0, rows_a, wts_a, out_a, osem_a, i > 0)

            @pl.when(i < N_CHUNKS // 2 - 1)
            def _():
                build(g0 + 2, idx_a, wts_a)
                fire(idx_a, rows_a, sem_a)

            wait(idx_b, rows_b, sem_b)
            compute_store(g0 + 1, rows_b, wts_b, out_b, osem_b, i > 0)
            return carry

        lax.fori_loop(0, N_CHUNKS // 2, pair_body, 0)
        pltpu.make_async_copy(out_a, _oslice(N_CHUNKS - 2), osem_a).wait()
        pltpu.make_async_copy(out_b, _oslice(N_CHUNKS - 1), osem_b).wait()

    return sc_gather


_ROWS_BLK = 2048


def _tc_body(pf_ref, samp_ref, wp_ref, wg_ref, b_ref, gamma_ref, beta_ref,
             out_ref):
    y = jnp.dot(pf_ref[...].astype(jnp.bfloat16), wp_ref[...],
                preferred_element_type=jnp.float32)
    y = y + jnp.dot(samp_ref[...].astype(jnp.bfloat16), wg_ref[...],
                    preferred_element_type=jnp.float32)
    y = y + b_ref[...]
    mu = jnp.mean(y, axis=1, keepdims=True)
    yc = y - mu
    var = jnp.mean(yc * yc, axis=1, keepdims=True)
    out_ref[...] = yc * lax.rsqrt(var + 1e-5) * gamma_ref[...] + beta_ref[...]


def _tc_fused(pf, samp, wp, wg, b, gamma, beta):
    n = pf.shape[0]
    n_blocks = pl.cdiv(n, _ROWS_BLK)
    return pl.pallas_call(
        _tc_body,
        grid=(n_blocks,),
        in_specs=[
            pl.BlockSpec((_ROWS_BLK, POINT_C), lambda i: (i, 0)),
            pl.BlockSpec((_ROWS_BLK, GRID_C), lambda i: (i, 0)),
            pl.BlockSpec((POINT_C, OUT_C), lambda i: (0, 0)),
            pl.BlockSpec((GRID_C, OUT_C), lambda i: (0, 0)),
            pl.BlockSpec((1, OUT_C), lambda i: (0, 0)),
            pl.BlockSpec((1, OUT_C), lambda i: (0, 0)),
            pl.BlockSpec((1, OUT_C), lambda i: (0, 0)),
        ],
        out_specs=pl.BlockSpec((_ROWS_BLK, OUT_C), lambda i: (i, 0)),
        out_shape=jax.ShapeDtypeStruct((n, OUT_C), jnp.float32),
    )(pf, samp, wp, wg, b, gamma, beta)


def kernel(grid_features, vertices, point_feats, W, b, gamma, beta):
    n = vertices.shape[0]
    table = grid_features[0].reshape(GRID_C, RES * RES * RES).T
    table_w = lax.bitcast_convert_type(
        table.astype(jnp.bfloat16).reshape(-1, WORDS, 2), jnp.float32)
    vflat = jnp.pad(vertices, ((0, N_PAD - n), (0, 0))).T.reshape(-1)
    wb = W.astype(jnp.bfloat16)
    perm = jnp.asarray(list(range(0, GRID_C, 2)) + list(range(1, GRID_C, 2)))
    wg = wb[POINT_C:][perm]
    samp_w = _make_sc_gather()(table_w, vflat)
    return _tc_fused(point_feats, samp_w, wb[:POINT_C], wg,
                     b[None, :], gamma[None, :], beta[None, :])


# speedup vs baseline: 1.3096x; 1.0262x over previous
import functools

import jax
import jax.numpy as jnp
from jax import lax
from jax.experimental import pallas as pl
from jax.experimental.pallas import tpu as pltpu
from jax.experimental.pallas import tpu_sc as plsc

RES = 32
GRID_C = 256
POINT_C = 128
OUT_C = 512

N_PAD = 100352
PER_WORKER = N_PAD // 32
CHUNK = 16
N_CHUNKS = PER_WORKER // CHUNK
WORDS = GRID_C // 2

_CORNER_OFF = (0, 1, 32, 33, 1024, 1025, 1056, 1057)


def _floor_clamp(t):
    i = t.astype(jnp.int32)
    f = i.astype(jnp.float32)
    i = jnp.where(f > t, i - 1, i)
    return jnp.minimum(jnp.maximum(i, 0), RES - 2)


@functools.cache
def _make_sc_gather(npts=N_PAD):
    mesh = plsc.VectorSubcoreMesh(core_axis_name="c", subcore_axis_name="s")
    PER_WORKER = npts // 32
    N_CHUNKS = PER_WORKER // CHUNK
    assert N_CHUNKS % 2 == 0 and (PER_WORKER * 3) % 8 == 0

    @functools.partial(
        pl.kernel,
        mesh=mesh,
        out_type=jax.ShapeDtypeStruct((npts, GRID_C), jnp.float32),
        scratch_types=[
            pltpu.VMEM((PER_WORKER,), jnp.float32),
            pltpu.VMEM((PER_WORKER,), jnp.float32),
            pltpu.VMEM((PER_WORKER,), jnp.float32),
            pltpu.VMEM((8 * CHUNK,), jnp.int32),
            pltpu.VMEM((8 * CHUNK,), jnp.int32),
            pltpu.VMEM((8 * CHUNK,), jnp.float32),
            pltpu.VMEM((8 * CHUNK,), jnp.float32),
            pltpu.VMEM((8 * CHUNK, WORDS), jnp.float32),
            pltpu.VMEM((8 * CHUNK, WORDS), jnp.float32),
            pltpu.VMEM((CHUNK, GRID_C), jnp.float32),
            pltpu.VMEM((CHUNK, GRID_C), jnp.float32),
            pltpu.SemaphoreType.DMA,
            pltpu.SemaphoreType.DMA,
            pltpu.SemaphoreType.DMA,
            pltpu.SemaphoreType.DMA,
        ],
    )
    def sc_gather(table_hbm, vflat_hbm, out_hbm, xbuf, ybuf,
                  zbuf, idx_a, idx_b, wts_a, wts_b, rows_a, rows_b, out_a,
                  out_b, sem_a, sem_b, osem_a, osem_b):
        cid = lax.axis_index("c")
        sid = lax.axis_index("s")
        wid = sid * 2 + cid
        base = pl.multiple_of(wid * PER_WORKER, 8)
        pltpu.sync_copy(vflat_hbm.at[pl.ds(base, PER_WORKER)], xbuf)
        pltpu.sync_copy(
            vflat_hbm.at[pl.ds(pl.multiple_of(npts + base, 8), PER_WORKER)],
            ybuf)
        pltpu.sync_copy(
            vflat_hbm.at[pl.ds(pl.multiple_of(2 * npts + base, 8),
                               PER_WORKER)],
            zbuf)

        zeros16 = jnp.zeros((16,), jnp.int32)
        gdn = lax.GatherDimensionNumbers(
            offset_dims=(), collapsed_slice_dims=(0,), start_index_map=(0,))

        def build(g, idx_v, wts_v):
            off = pl.multiple_of(g * CHUNK, 8)
            vx = xbuf[pl.ds(off, CHUNK)]
            vy = ybuf[pl.ds(off, CHUNK)]
            vz = zbuf[pl.ds(off, CHUNK)]

            x31 = vx * jnp.float32(RES - 1)
            y31 = vy * jnp.float32(RES - 1)
            z31 = vz * jnp.float32(RES - 1)
            x0 = _floor_clamp(x31)
            y0 = _floor_clamp(y31)
            z0 = _floor_clamp(z31)
            wx1 = x31 - x0.astype(jnp.float32)
            wy1 = y31 - y0.astype(jnp.float32)
            wz1 = z31 - z0.astype(jnp.float32)
            wx0 = 1.0 - wx1
            wy0 = 1.0 - wy1
            wz0 = 1.0 - wz1

            w00 = wz0 * wy0
            w01 = wz0 * wy1
            w10 = wz1 * wy0
            w11 = wz1 * wy1
            for ci, w in enumerate((w00 * wx0, w00 * wx1, w01 * wx0,
                                    w01 * wx1, w10 * wx0, w10 * wx1,
                                    w11 * wx0, w11 * wx1)):
                wts_v[pl.ds(ci * CHUNK, CHUNK)] = w

            r = z0 * (RES * RES) + y0 * RES + x0
            for ci in range(8):
                idx_v[pl.ds(ci * CHUNK, CHUNK)] = r + _CORNER_OFF[ci]

        def fire(idx_v, rows_v, sem):
            pltpu.async_copy(table_hbm.at[idx_v], rows_v, sem)

        def wait(idx_v, rows_v, sem):
            pltpu.make_async_copy(table_hbm.at[idx_v], rows_v, sem).wait()

        def _oslice(g):
            return out_hbm.at[pl.ds(pl.multiple_of(base + g * CHUNK, 8),
                                    CHUNK)]

        def compute_store(g, rows_v, wts_v, out_v, osem, drain_pred):
            wrows = [wts_v[pl.ds(ci * CHUNK, CHUNK)] for ci in range(8)]
            i32 = jnp.int32
            f32 = jnp.float32

            @pl.when(drain_pred)
            def _():
                pltpu.make_async_copy(out_v, _oslice(g), osem).wait()

            @plsc.parallel_loop(0, CHUNK, unroll=2)
            def pt_body(l):
                lsplat = (zeros16 + l)[:, None]
                wvec = [lax.gather(
                            wrows[ci], lsplat, gdn, (1,),
                            mode=lax.GatherScatterMode.PROMISE_IN_BOUNDS)
                        for ci in range(8)]
                for sgi in range(WORDS // 16):
                    acc_a = None
                    acc_b = None
                    for ci in range(8):
                        seg = rows_v[ci * CHUNK + l, pl.ds(sgi * 16, 16)]
                        w_i = lax.bitcast_convert_type(seg, i32)
                        a = lax.bitcast_convert_type(
                            lax.shift_left(w_i, 16), f32)
                        bb = seg
                        ta = a * wvec[ci]
                        tb = bb * wvec[ci]
                        acc_a = ta if acc_a is None else acc_a + ta
                        acc_b = tb if acc_b is None else acc_b + tb
                    out_v[l, pl.ds(sgi * 16, 16)] = acc_a
                    out_v[l, pl.ds(WORDS + sgi * 16, 16)] = acc_b

            pltpu.async_copy(out_v, _oslice(g), osem)

        build(0, idx_a, wts_a)
        fire(idx_a, rows_a, sem_a)

        def pair_body(i, carry):
            g0 = i * 2
            build(g0 + 1, idx_b, wts_b)
            fire(idx_b, rows_b, sem_b)
            wait(idx_a, rows_a, sem_a)
            compute_store(g0, rows_a, wts_a, out_a, osem_a, i > 0)

            @pl.when(i < N_CHUNKS // 2 - 1)
            def _():
                build(g0 + 2, idx_a, wts_a)
                fire(idx_a, rows_a, sem_a)

            wait(idx_b, rows_b, sem_b)
            compute_store(g0 + 1, rows_b, wts_b, out_b, osem_b, i > 0)
            return carry

        lax.fori_loop(0, N_CHUNKS // 2, pair_body, 0)
        pltpu.make_async_copy(out_a, _oslice(N_CHUNKS - 2), osem_a).wait()
        pltpu.make_async_copy(out_b, _oslice(N_CHUNKS - 1), osem_b).wait()

    return sc_gather


_ROWS_BLK = 4096


def _tc_body(pf_ref, samp_ref, wp_ref, wg_ref, b_ref, gamma_ref, beta_ref,
             out_ref):
    y = jnp.dot(pf_ref[...].astype(jnp.bfloat16), wp_ref[...],
                preferred_element_type=jnp.float32)
    y = y + jnp.dot(samp_ref[...].astype(jnp.bfloat16), wg_ref[...],
                    preferred_element_type=jnp.float32)
    y = y + b_ref[...]
    mu = jnp.mean(y, axis=1, keepdims=True)
    yc = y - mu
    var = jnp.mean(yc * yc, axis=1, keepdims=True)
    out_ref[...] = yc * lax.rsqrt(var + 1e-5) * gamma_ref[...] + beta_ref[...]


def _tc_fused(pf, samp, wp, wg, b, gamma, beta):
    n = pf.shape[0]
    n_blocks = pl.cdiv(n, _ROWS_BLK)
    return pl.pallas_call(
        _tc_body,
        grid=(n_blocks,),
        in_specs=[
            pl.BlockSpec((_ROWS_BLK, POINT_C), lambda i: (i, 0)),
            pl.BlockSpec((_ROWS_BLK, GRID_C), lambda i: (i, 0)),
            pl.BlockSpec((POINT_C, OUT_C), lambda i: (0, 0)),
            pl.BlockSpec((GRID_C, OUT_C), lambda i: (0, 0)),
            pl.BlockSpec((1, OUT_C), lambda i: (0, 0)),
            pl.BlockSpec((1, OUT_C), lambda i: (0, 0)),
            pl.BlockSpec((1, OUT_C), lambda i: (0, 0)),
        ],
        out_specs=pl.BlockSpec((_ROWS_BLK, OUT_C), lambda i: (i, 0)),
        out_shape=jax.ShapeDtypeStruct((n, OUT_C), jnp.float32),
    )(pf, samp, wp, wg, b, gamma, beta)


def kernel(grid_features, vertices, point_feats, W, b, gamma, beta):
    n = vertices.shape[0]
    table = grid_features[0].reshape(GRID_C, RES * RES * RES).T
    table_w = lax.bitcast_convert_type(
        table.astype(jnp.bfloat16).reshape(-1, WORDS, 2), jnp.float32)
    vflat = jnp.pad(vertices, ((0, N_PAD - n), (0, 0))).T.reshape(-1)
    wb = W.astype(jnp.bfloat16)
    perm = jnp.asarray(list(range(0, GRID_C, 2)) + list(range(1, GRID_C, 2)))
    wg = wb[POINT_C:][perm]
    samp_w = _make_sc_gather()(table_w, vflat)
    return _tc_fused(point_feats, samp_w, wb[:POINT_C], wg,
                     b[None, :], gamma[None, :], beta[None, :])
